# Initial kernel scaffold; baseline (speedup 1.0000x reference)
import jax
import jax.numpy as jnp
from jax.experimental import pallas as pl


def kernel(x, pos, edge_index, W11, W12, g1, b1, W21, W22, g2, b2):
    raise NotImplementedError("write your pallas kernel here")


# speedup vs baseline: 4.1992x; 4.1992x over previous
import functools

import jax
import jax.numpy as jnp
from jax import lax
from jax.experimental import pallas as pl
from jax.experimental.pallas import tpu as pltpu
from jax.experimental.pallas import tpu_sc as plsc

N = 10000
E = 320000
D = 128
N2 = 10240
NW = 32
RNG = N2 // NW
CAP = 16384
SCAN_WIN = 2048
E2 = ((E + SCAN_WIN - 1) // SCAN_WIN) * SCAN_WIN
GW = 128
NB = N2 // 1280

_mesh = plsc.VectorSubcoreMesh(core_axis_name="c", subcore_axis_name="s")
_sc_params = pltpu.CompilerParams(needs_layout_passes=False)


@functools.partial(
    pl.kernel,
    out_type=(
        jax.ShapeDtypeStruct((NW, CAP), jnp.int32),
        jax.ShapeDtypeStruct((NW, CAP), jnp.int32),
        jax.ShapeDtypeStruct((NW, 16), jnp.int32),
    ),
    mesh=_mesh,
    compiler_params=_sc_params,
    scratch_types=[
        pltpu.VMEM((SCAN_WIN,), jnp.int32),
        pltpu.VMEM((SCAN_WIN,), jnp.int32),
        pltpu.VMEM((CAP,), jnp.int32),
        pltpu.VMEM((CAP,), jnp.int32),
        pltpu.VMEM((16,), jnp.int32),
    ],
)
def _select(src_hbm, dst_hbm, sel_hbm, selsrc_hbm, cnt_hbm,
            srcw, dstw, sel, selsrc, cntv):
    wid = lax.axis_index("s") * 2 + lax.axis_index("c")
    lo = wid * RNG

    def win_body(win, cnt):
        pltpu.sync_copy(src_hbm.at[pl.ds(win * SCAN_WIN, SCAN_WIN)], srcw)
        pltpu.sync_copy(dst_hbm.at[pl.ds(win * SCAN_WIN, SCAN_WIN)], dstw)

        def grp(i, c):
            s16 = srcw[pl.ds(i * 16, 16)]
            d16 = dstw[pl.ds(i * 16, 16)]
            dl = d16 - lo
            m = (dl >= 0) & (dl < RNG)
            packed = jnp.where(m, (s16 << 9) | dl, jnp.int32(0x7FFFFFFF))
            ps = jnp.sort(packed)
            sel[pl.ds(c, 16)] = ps
            selsrc[pl.ds(c, 16)] = jnp.where(ps == 0x7FFFFFFF, lo, ps >> 9)
            return c + plsc.all_reduce_population_count(m)[0]

        return lax.fori_loop(0, SCAN_WIN // 16, grp, cnt)

    cnt = lax.fori_loop(0, E2 // SCAN_WIN, win_body, 0)

    pad = jnp.full((16,), (lo << 9) | RNG, jnp.int32)
    pad_s = jnp.full((16,), lo, jnp.int32)
    for k in range(GW // 16):
        sel[pl.ds(cnt + k * 16, 16)] = pad
        selsrc[pl.ds(cnt + k * 16, 16)] = pad_s
    cnt_r = ((cnt + GW - 1) // GW) * GW

    cntv[...] = jnp.full((16,), 0, jnp.int32) + cnt_r
    pltpu.sync_copy(cntv, cnt_hbm.at[wid])
    pltpu.sync_copy(sel, sel_hbm.at[wid])
    pltpu.sync_copy(selsrc, selsrc_hbm.at[wid])


@functools.partial(
    pl.kernel,
    out_type=jax.ShapeDtypeStruct((N2, D), jnp.float32),
    mesh=_mesh,
    compiler_params=_sc_params,
    scratch_types=[
        pltpu.VMEM((CAP,), jnp.int32),
        pltpu.VMEM((CAP,), jnp.int32),
        pltpu.VMEM((16,), jnp.int32),
        pltpu.VMEM((RNG + 8, D), jnp.float32),
        pltpu.VMEM((GW, D), jnp.float32),
        pltpu.SemaphoreType.DMA,
    ],
)
def _segmax(a_hbm, sel_hbm, selsrc_hbm, cnt_hbm, out_hbm,
            sell, srcl, cntv, acc, rows, sem):
    wid = lax.axis_index("s") * 2 + lax.axis_index("c")
    n0 = wid * RNG
    pltpu.sync_copy(sel_hbm.at[wid], sell)
    pltpu.sync_copy(selsrc_hbm.at[wid], srcl)
    pltpu.sync_copy(cnt_hbm.at[wid], cntv)
    pltpu.sync_copy(a_hbm.at[pl.ds(n0, RNG)], acc.at[pl.ds(0, RNG)])
    nwin = cntv[...][0] // GW

    def win_body(win, carry):
        pltpu.async_copy(a_hbm.at[srcl.at[pl.ds(win * GW, GW)]], rows,
                         sem).wait()

        def grp_body(g, c2):
            d16 = sell[pl.ds(win * GW + g * 16, 16)] & 511
            for j in range(16):
                dloc = d16[j]
                for f in range(D // 16):
                    sl = pl.ds(f * 16, 16)
                    acc[dloc, sl] = jnp.maximum(acc[dloc, sl], rows[g * 16 + j, sl])
            return c2

        return lax.fori_loop(0, GW // 16, grp_body, carry)

    lax.fori_loop(0, nwin, win_body, 0)
    pltpu.sync_copy(acc.at[pl.ds(0, RNG)], out_hbm.at[pl.ds(n0, RNG)])


def _lin_body(xc_ref, w_ref, o_ref):
    o_ref[...] = jnp.dot(xc_ref[...], w_ref[...],
                         preferred_element_type=jnp.float32)


_lin = pl.pallas_call(
    _lin_body,
    grid=(NB,),
    in_specs=[
        pl.BlockSpec((1280, 2 * D), lambda i: (i, 0)),
        pl.BlockSpec((2 * D, 2 * D), lambda i: (0, 0)),
    ],
    out_specs=pl.BlockSpec((1280, 2 * D), lambda i: (i, 0)),
    out_shape=jax.ShapeDtypeStruct((N2, 2 * D), jnp.float32),
)


def _mid_body(s_ref, p_ref, w_ref, y_ref, sm_ref, sq_ref):
    t = s_ref[...] - p_ref[...]
    y = jnp.dot(t, w_ref[...], preferred_element_type=jnp.float32)
    rows = jax.lax.broadcasted_iota(jnp.int32, y.shape, 0) + pl.program_id(0) * 1280
    y = jnp.where(rows < N, y, 0.0)
    y_ref[...] = y
    sm_ref[...] = jnp.sum(y, axis=0, keepdims=True)[None]
    sq_ref[...] = jnp.sum(y * y, axis=0, keepdims=True)[None]


_mid = pl.pallas_call(
    _mid_body,
    grid=(NB,),
    in_specs=[
        pl.BlockSpec((1280, D), lambda i: (i, 0)),
        pl.BlockSpec((1280, D), lambda i: (i, 0)),
        pl.BlockSpec((D, D), lambda i: (0, 0)),
    ],
    out_specs=[
        pl.BlockSpec((1280, D), lambda i: (i, 0)),
        pl.BlockSpec((1, 1, D), lambda i: (i, 0, 0)),
        pl.BlockSpec((1, 1, D), lambda i: (i, 0, 0)),
    ],
    out_shape=[
        jax.ShapeDtypeStruct((N2, D), jnp.float32),
        jax.ShapeDtypeStruct((NB, 1, D), jnp.float32),
        jax.ShapeDtypeStruct((NB, 1, D), jnp.float32),
    ],
)


def _apply1_body(y_ref, sm_ref, sq_ref, g_ref, b_ref, pos2f_ref,
                 w21a_ref, w21bp_ref, a2_ref, p2_ref):
    m = jnp.sum(sm_ref[...], axis=0) / N
    v = jnp.sum(sq_ref[...], axis=0) / N - m * m
    s = g_ref[...] * jax.lax.rsqrt(v + 1e-5)
    h = jnp.maximum(y_ref[...] * s + (b_ref[...] - m * s), 0.0)
    p2 = jnp.dot(pos2f_ref[...], w21bp_ref[...],
                 preferred_element_type=jnp.float32)
    a2_ref[...] = jnp.dot(h, w21a_ref[...],
                          preferred_element_type=jnp.float32) + p2
    p2_ref[...] = p2


_apply1 = pl.pallas_call(
    _apply1_body,
    grid=(NB,),
    in_specs=[
        pl.BlockSpec((1280, D), lambda i: (i, 0)),
        pl.BlockSpec((NB, 1, D), lambda i: (0, 0, 0)),
        pl.BlockSpec((NB, 1, D), lambda i: (0, 0, 0)),
        pl.BlockSpec((1, D), lambda i: (0, 0)),
        pl.BlockSpec((1, D), lambda i: (0, 0)),
        pl.BlockSpec((1280, D), lambda i: (i, 0)),
        pl.BlockSpec((D, D), lambda i: (0, 0)),
        pl.BlockSpec((D, D), lambda i: (0, 0)),
    ],
    out_specs=[
        pl.BlockSpec((1280, D), lambda i: (i, 0)),
        pl.BlockSpec((1280, D), lambda i: (i, 0)),
    ],
    out_shape=[
        jax.ShapeDtypeStruct((N2, D), jnp.float32),
        jax.ShapeDtypeStruct((N2, D), jnp.float32),
    ],
)


def _apply2_body(y_ref, sm_ref, sq_ref, g_ref, b_ref, x_ref, o_ref):
    m = jnp.sum(sm_ref[...], axis=0) / N
    v = jnp.sum(sq_ref[...], axis=0) / N - m * m
    s = g_ref[...] * jax.lax.rsqrt(v + 1e-5)
    o_ref[...] = jnp.maximum(y_ref[...] * s + (b_ref[...] - m * s) + x_ref[...],
                             0.0)


_apply2 = pl.pallas_call(
    _apply2_body,
    grid=(NB,),
    in_specs=[
        pl.BlockSpec((1280, D), lambda i: (i, 0)),
        pl.BlockSpec((NB, 1, D), lambda i: (0, 0, 0)),
        pl.BlockSpec((NB, 1, D), lambda i: (0, 0, 0)),
        pl.BlockSpec((1, D), lambda i: (0, 0)),
        pl.BlockSpec((1, D), lambda i: (0, 0)),
        pl.BlockSpec((1280, D), lambda i: (i, 0)),
    ],
    out_specs=pl.BlockSpec((1280, D), lambda i: (i, 0)),
    out_shape=jax.ShapeDtypeStruct((N2, D), jnp.float32),
)


def kernel(x, pos, edge_index, W11, W12, g1, b1, W21, W22, g2, b2):
    pos2 = pos[:, :2]
    xc = jnp.zeros((N2, 2 * D), jnp.float32)
    xc = xc.at[:N, :D].set(x).at[:N, D:D + 2].set(pos2)
    pos2f = jnp.zeros((N2, D), jnp.float32).at[:N, :2].set(pos2)
    xpad = jnp.zeros((N2, D), jnp.float32).at[:N].set(x)
    Wc1 = jnp.zeros((2 * D, 2 * D), jnp.float32)
    Wc1 = Wc1.at[:D + 2, :D].set(W11).at[D:D + 2, D:].set(W11[D:])
    w21bp = jnp.zeros((D, D), jnp.float32).at[:2].set(W21[D:])
    w21a = W21[:D]
    g1r, b1r = g1.reshape(1, D), b1.reshape(1, D)
    g2r, b2r = g2.reshape(1, D), b2.reshape(1, D)

    srcp = jnp.concatenate([edge_index[0],
                            jnp.zeros((E2 - E,), jnp.int32)])
    dstp = jnp.concatenate([edge_index[1],
                            jnp.full((E2 - E,), 1 << 20, jnp.int32)])

    sel, selsrc, cnts = _select(srcp, dstp)

    ap1 = _lin(xc, Wc1)
    a1 = ap1[:, :D]
    p1 = ap1[:, D:]
    s1 = _segmax(a1, sel, selsrc, cnts)
    y1, sm1, sq1 = _mid(s1, p1, W12)
    a2, p2 = _apply1(y1, sm1, sq1, g1r, b1r, pos2f, w21a, w21bp)
    s2 = _segmax(a2, sel, selsrc, cnts)
    y2, sm2, sq2 = _mid(s2, p2, W22)
    out = _apply2(y2, sm2, sq2, g2r, b2r, xpad)
    return out[:N]


# speedup vs baseline: 4.8003x; 1.1431x over previous
import functools

import jax
import jax.numpy as jnp
from jax import lax
from jax.experimental import pallas as pl
from jax.experimental.pallas import tpu as pltpu
from jax.experimental.pallas import tpu_sc as plsc

N = 10000
E = 320000
D = 128
N2 = 10240
NW = 32
RNG = N2 // NW
CAP = 16384
SCAN_WIN = 2048
E2 = ((E + SCAN_WIN - 1) // SCAN_WIN) * SCAN_WIN
GW = 128
NB = N2 // 1280

_mesh = plsc.VectorSubcoreMesh(core_axis_name="c", subcore_axis_name="s")
_sc_params = pltpu.CompilerParams(needs_layout_passes=False)


@functools.partial(
    pl.kernel,
    out_type=(
        jax.ShapeDtypeStruct((NW, CAP), jnp.int32),
        jax.ShapeDtypeStruct((NW, CAP), jnp.int32),
        jax.ShapeDtypeStruct((NW, 16), jnp.int32),
    ),
    mesh=_mesh,
    compiler_params=_sc_params,
    scratch_types=[
        pltpu.VMEM((SCAN_WIN,), jnp.int32),
        pltpu.VMEM((SCAN_WIN,), jnp.int32),
        pltpu.VMEM((CAP,), jnp.int32),
        pltpu.VMEM((CAP,), jnp.int32),
        pltpu.VMEM((16,), jnp.int32),
    ],
)
def _select(src_hbm, dst_hbm, sel_hbm, selsrc_hbm, cnt_hbm,
            srcw, dstw, sel, selsrc, cntv):
    wid = lax.axis_index("s") * 2 + lax.axis_index("c")
    lo = wid * RNG

    def win_body(win, cnt):
        pltpu.sync_copy(src_hbm.at[pl.ds(win * SCAN_WIN, SCAN_WIN)], srcw)
        pltpu.sync_copy(dst_hbm.at[pl.ds(win * SCAN_WIN, SCAN_WIN)], dstw)

        def grp4(i4, c):
            for u in range(4):
                i = i4 * 4 + u
                s16 = srcw[pl.ds(i * 16, 16)]
                d16 = dstw[pl.ds(i * 16, 16)]
                dl = d16 - lo
                m = (dl >= 0) & (dl < RNG)
                packed = jnp.where(m, (s16 << 9) | dl, jnp.int32(0x7FFFFFFF))
                ps = jnp.sort(packed)
                sel[pl.ds(c, 16)] = ps
                selsrc[pl.ds(c, 16)] = jnp.where(ps == 0x7FFFFFFF, lo, ps >> 9)
                c = c + plsc.all_reduce_population_count(m)[0]
            return c

        return lax.fori_loop(0, SCAN_WIN // 64, grp4, cnt)

    cnt = lax.fori_loop(0, E2 // SCAN_WIN, win_body, 0)

    pad = jnp.full((16,), (lo << 9) | RNG, jnp.int32)
    pad_s = jnp.full((16,), lo, jnp.int32)
    for k in range(2 * GW // 16):
        sel[pl.ds(cnt + k * 16, 16)] = pad
        selsrc[pl.ds(cnt + k * 16, 16)] = pad_s
    cnt_r = ((cnt + 2 * GW - 1) // (2 * GW)) * (2 * GW)

    cntv[...] = jnp.full((16,), 0, jnp.int32) + cnt_r
    pltpu.sync_copy(cntv, cnt_hbm.at[wid])
    pltpu.sync_copy(sel, sel_hbm.at[wid])
    pltpu.sync_copy(selsrc, selsrc_hbm.at[wid])


@functools.partial(
    pl.kernel,
    out_type=jax.ShapeDtypeStruct((N2, D), jnp.float32),
    mesh=_mesh,
    compiler_params=_sc_params,
    scratch_types=[
        pltpu.VMEM((CAP,), jnp.int32),
        pltpu.VMEM((CAP,), jnp.int32),
        pltpu.VMEM((16,), jnp.int32),
        pltpu.VMEM((RNG + 8, D), jnp.float32),
        pltpu.VMEM((GW, D), jnp.float32),
        pltpu.VMEM((GW, D), jnp.float32),
        pltpu.SemaphoreType.DMA,
        pltpu.SemaphoreType.DMA,
    ],
)
def _segmax(a_hbm, sel_hbm, selsrc_hbm, cnt_hbm, out_hbm,
            sell, srcl, cntv, acc, rows0, rows1, sem0, sem1):
    wid = lax.axis_index("s") * 2 + lax.axis_index("c")
    n0 = wid * RNG
    pltpu.sync_copy(sel_hbm.at[wid], sell)
    pltpu.sync_copy(selsrc_hbm.at[wid], srcl)
    pltpu.sync_copy(cnt_hbm.at[wid], cntv)
    pltpu.sync_copy(a_hbm.at[pl.ds(n0, RNG)], acc.at[pl.ds(0, RNG)])
    nwin2 = cntv[...][0] // (2 * GW)

    def compute(rows, base):
        def grp_body(g, c2):
            d16 = sell[pl.ds(base + g * 16, 16)] & 511
            for j in range(16):
                dloc = d16[j]
                for f in range(D // 16):
                    sl = pl.ds(f * 16, 16)
                    acc[dloc, sl] = jnp.maximum(acc[dloc, sl],
                                                rows[g * 16 + j, sl])
            return c2

        lax.fori_loop(0, GW // 16, grp_body, 0)

    pltpu.async_copy(a_hbm.at[srcl.at[pl.ds(0, GW)]], rows0, sem0)

    def win_body(k, carry):
        base = k * 2 * GW
        pltpu.async_copy(a_hbm.at[srcl.at[pl.ds(base + GW, GW)]], rows1, sem1)
        pltpu.make_async_copy(a_hbm.at[pl.ds(0, GW)], rows0, sem0).wait()
        compute(rows0, base)

        @pl.when(k + 1 < nwin2)
        def _():
            pltpu.async_copy(a_hbm.at[srcl.at[pl.ds(base + 2 * GW, GW)]],
                             rows0, sem0)

        pltpu.make_async_copy(a_hbm.at[pl.ds(0, GW)], rows1, sem1).wait()
        compute(rows1, base + GW)
        return carry

    lax.fori_loop(0, nwin2, win_body, 0)
    pltpu.sync_copy(acc.at[pl.ds(0, RNG)], out_hbm.at[pl.ds(n0, RNG)])


def _lin_body(xc_ref, w_ref, o_ref):
    o_ref[...] = jnp.dot(xc_ref[...], w_ref[...],
                         preferred_element_type=jnp.float32)


_lin = pl.pallas_call(
    _lin_body,
    grid=(NB,),
    in_specs=[
        pl.BlockSpec((1280, 2 * D), lambda i: (i, 0)),
        pl.BlockSpec((2 * D, 2 * D), lambda i: (0, 0)),
    ],
    out_specs=pl.BlockSpec((1280, 2 * D), lambda i: (i, 0)),
    out_shape=jax.ShapeDtypeStruct((N2, 2 * D), jnp.float32),
)


def _mid_body(s_ref, p_ref, w_ref, y_ref, sm_ref, sq_ref):
    t = s_ref[...] - p_ref[...]
    y = jnp.dot(t, w_ref[...], preferred_element_type=jnp.float32)
    rows = jax.lax.broadcasted_iota(jnp.int32, y.shape, 0) + pl.program_id(0) * 1280
    y = jnp.where(rows < N, y, 0.0)
    y_ref[...] = y
    sm_ref[...] = jnp.sum(y, axis=0, keepdims=True)[None]
    sq_ref[...] = jnp.sum(y * y, axis=0, keepdims=True)[None]


_mid = pl.pallas_call(
    _mid_body,
    grid=(NB,),
    in_specs=[
        pl.BlockSpec((1280, D), lambda i: (i, 0)),
        pl.BlockSpec((1280, D), lambda i: (i, 0)),
        pl.BlockSpec((D, D), lambda i: (0, 0)),
    ],
    out_specs=[
        pl.BlockSpec((1280, D), lambda i: (i, 0)),
        pl.BlockSpec((1, 1, D), lambda i: (i, 0, 0)),
        pl.BlockSpec((1, 1, D), lambda i: (i, 0, 0)),
    ],
    out_shape=[
        jax.ShapeDtypeStruct((N2, D), jnp.float32),
        jax.ShapeDtypeStruct((NB, 1, D), jnp.float32),
        jax.ShapeDtypeStruct((NB, 1, D), jnp.float32),
    ],
)


def _apply1_body(y_ref, sm_ref, sq_ref, g_ref, b_ref, pos2f_ref,
                 w21a_ref, w21bp_ref, a2_ref, p2_ref):
    m = jnp.sum(sm_ref[...], axis=0) / N
    v = jnp.sum(sq_ref[...], axis=0) / N - m * m
    s = g_ref[...] * jax.lax.rsqrt(v + 1e-5)
    h = jnp.maximum(y_ref[...] * s + (b_ref[...] - m * s), 0.0)
    p2 = jnp.dot(pos2f_ref[...], w21bp_ref[...],
                 preferred_element_type=jnp.float32)
    a2_ref[...] = jnp.dot(h, w21a_ref[...],
                          preferred_element_type=jnp.float32) + p2
    p2_ref[...] = p2


_apply1 = pl.pallas_call(
    _apply1_body,
    grid=(NB,),
    in_specs=[
        pl.BlockSpec((1280, D), lambda i: (i, 0)),
        pl.BlockSpec((NB, 1, D), lambda i: (0, 0, 0)),
        pl.BlockSpec((NB, 1, D), lambda i: (0, 0, 0)),
        pl.BlockSpec((1, D), lambda i: (0, 0)),
        pl.BlockSpec((1, D), lambda i: (0, 0)),
        pl.BlockSpec((1280, D), lambda i: (i, 0)),
        pl.BlockSpec((D, D), lambda i: (0, 0)),
        pl.BlockSpec((D, D), lambda i: (0, 0)),
    ],
    out_specs=[
        pl.BlockSpec((1280, D), lambda i: (i, 0)),
        pl.BlockSpec((1280, D), lambda i: (i, 0)),
    ],
    out_shape=[
        jax.ShapeDtypeStruct((N2, D), jnp.float32),
        jax.ShapeDtypeStruct((N2, D), jnp.float32),
    ],
)


def _apply2_body(y_ref, sm_ref, sq_ref, g_ref, b_ref, x_ref, o_ref):
    m = jnp.sum(sm_ref[...], axis=0) / N
    v = jnp.sum(sq_ref[...], axis=0) / N - m * m
    s = g_ref[...] * jax.lax.rsqrt(v + 1e-5)
    o_ref[...] = jnp.maximum(y_ref[...] * s + (b_ref[...] - m * s) + x_ref[...],
                             0.0)


_apply2 = pl.pallas_call(
    _apply2_body,
    grid=(NB,),
    in_specs=[
        pl.BlockSpec((1280, D), lambda i: (i, 0)),
        pl.BlockSpec((NB, 1, D), lambda i: (0, 0, 0)),
        pl.BlockSpec((NB, 1, D), lambda i: (0, 0, 0)),
        pl.BlockSpec((1, D), lambda i: (0, 0)),
        pl.BlockSpec((1, D), lambda i: (0, 0)),
        pl.BlockSpec((1280, D), lambda i: (i, 0)),
    ],
    out_specs=pl.BlockSpec((1280, D), lambda i: (i, 0)),
    out_shape=jax.ShapeDtypeStruct((N2, D), jnp.float32),
)


def kernel(x, pos, edge_index, W11, W12, g1, b1, W21, W22, g2, b2):
    pos2 = pos[:, :2]
    xc = jnp.zeros((N2, 2 * D), jnp.float32)
    xc = xc.at[:N, :D].set(x).at[:N, D:D + 2].set(pos2)
    pos2f = jnp.zeros((N2, D), jnp.float32).at[:N, :2].set(pos2)
    xpad = jnp.zeros((N2, D), jnp.float32).at[:N].set(x)
    Wc1 = jnp.zeros((2 * D, 2 * D), jnp.float32)
    Wc1 = Wc1.at[:D + 2, :D].set(W11).at[D:D + 2, D:].set(W11[D:])
    w21bp = jnp.zeros((D, D), jnp.float32).at[:2].set(W21[D:])
    w21a = W21[:D]
    g1r, b1r = g1.reshape(1, D), b1.reshape(1, D)
    g2r, b2r = g2.reshape(1, D), b2.reshape(1, D)

    srcp = jnp.concatenate([edge_index[0],
                            jnp.zeros((E2 - E,), jnp.int32)])
    dstp = jnp.concatenate([edge_index[1],
                            jnp.full((E2 - E,), 1 << 20, jnp.int32)])

    sel, selsrc, cnts = _select(srcp, dstp)

    ap1 = _lin(xc, Wc1)
    a1 = ap1[:, :D]
    p1 = ap1[:, D:]
    s1 = _segmax(a1, sel, selsrc, cnts)
    y1, sm1, sq1 = _mid(s1, p1, W12)
    a2, p2 = _apply1(y1, sm1, sq1, g1r, b1r, pos2f, w21a, w21bp)
    s2 = _segmax(a2, sel, selsrc, cnts)
    y2, sm2, sq2 = _mid(s2, p2, W22)
    out = _apply2(y2, sm2, sq2, g2r, b2r, xpad)
    return out[:N]


# speedup vs baseline: 5.8029x; 1.2089x over previous
import functools

import jax
import jax.numpy as jnp
from jax import lax
from jax.experimental import pallas as pl
from jax.experimental.pallas import tpu as pltpu
from jax.experimental.pallas import tpu_sc as plsc

N = 10000
E = 320000
D = 128
N2 = 10240
NW = 32
RNG = N2 // NW
CAP = 16384
SCAN_WIN = 2048
E2 = ((E + 2 * SCAN_WIN - 1) // (2 * SCAN_WIN)) * (2 * SCAN_WIN)
GW = 128
NB = N2 // 1280

_mesh = plsc.VectorSubcoreMesh(core_axis_name="c", subcore_axis_name="s")
_sc_params = pltpu.CompilerParams(needs_layout_passes=False)


@functools.partial(
    pl.kernel,
    out_type=(
        jax.ShapeDtypeStruct((NW, CAP), jnp.int32),
        jax.ShapeDtypeStruct((NW, 16), jnp.int32),
    ),
    mesh=_mesh,
    compiler_params=_sc_params,
    scratch_types=[
        pltpu.VMEM((SCAN_WIN,), jnp.int32),
        pltpu.VMEM((SCAN_WIN,), jnp.int32),
        pltpu.VMEM((SCAN_WIN,), jnp.int32),
        pltpu.VMEM((SCAN_WIN,), jnp.int32),
        pltpu.VMEM((CAP,), jnp.int32),
        pltpu.VMEM((16,), jnp.int32),
        pltpu.SemaphoreType.DMA,
        pltpu.SemaphoreType.DMA,
    ],
)
def _select(src_hbm, dst_hbm, sel_hbm, cnt_hbm,
            srcw0, dstw0, srcw1, dstw1, sel, cntv, sem0, sem1):
    wid = lax.axis_index("s") * 2 + lax.axis_index("c")
    lo = wid * RNG
    nwin = E2 // SCAN_WIN

    def scan(srcw, dstw, cnt):
        def grp4(i4, c):
            for u in range(4):
                i = i4 * 4 + u
                s16 = srcw[pl.ds(i * 16, 16)]
                d16 = dstw[pl.ds(i * 16, 16)]
                dl = d16 - lo
                m = (dl >= 0) & (dl < RNG)
                packed = jnp.where(m, (s16 << 9) | dl, jnp.int32(0x7FFFFFFF))
                ps = jnp.sort(packed)
                sel[pl.ds(c, 16)] = ps
                c = c + plsc.all_reduce_population_count(m)[0]
            return c

        return lax.fori_loop(0, SCAN_WIN // 64, grp4, cnt)

    pltpu.async_copy(src_hbm.at[pl.ds(0, SCAN_WIN)], srcw0, sem0)
    pltpu.async_copy(dst_hbm.at[pl.ds(0, SCAN_WIN)], dstw0, sem0)

    def win2_body(w2, cnt):
        base = w2 * 2
        pltpu.async_copy(src_hbm.at[pl.ds((base + 1) * SCAN_WIN, SCAN_WIN)],
                         srcw1, sem1)
        pltpu.async_copy(dst_hbm.at[pl.ds((base + 1) * SCAN_WIN, SCAN_WIN)],
                         dstw1, sem1)
        pltpu.make_async_copy(src_hbm.at[pl.ds(0, SCAN_WIN)], srcw0, sem0).wait()
        pltpu.make_async_copy(src_hbm.at[pl.ds(0, SCAN_WIN)], dstw0, sem0).wait()
        cnt = scan(srcw0, dstw0, cnt)

        @pl.when(base + 2 < nwin)
        def _():
            pltpu.async_copy(src_hbm.at[pl.ds((base + 2) * SCAN_WIN, SCAN_WIN)],
                             srcw0, sem0)
            pltpu.async_copy(dst_hbm.at[pl.ds((base + 2) * SCAN_WIN, SCAN_WIN)],
                             dstw0, sem0)

        pltpu.make_async_copy(src_hbm.at[pl.ds(0, SCAN_WIN)], srcw1, sem1).wait()
        pltpu.make_async_copy(src_hbm.at[pl.ds(0, SCAN_WIN)], dstw1, sem1).wait()
        return scan(srcw1, dstw1, cnt)

    cnt = lax.fori_loop(0, nwin // 2, win2_body, 0)

    pad = jnp.full((16,), (lo << 9) | RNG, jnp.int32)
    for k in range(2 * GW // 16):
        sel[pl.ds(cnt + k * 16, 16)] = pad
    cnt_r = ((cnt + 2 * GW - 1) // (2 * GW)) * (2 * GW)

    cntv[...] = jnp.full((16,), 0, jnp.int32) + cnt_r
    pltpu.sync_copy(cntv, cnt_hbm.at[wid])
    pltpu.sync_copy(sel, sel_hbm.at[wid])


@functools.partial(
    pl.kernel,
    out_type=jax.ShapeDtypeStruct((N2, D), jnp.float32),
    mesh=_mesh,
    compiler_params=_sc_params,
    scratch_types=[
        pltpu.VMEM((CAP,), jnp.int32),
        pltpu.VMEM((CAP,), jnp.int32),
        pltpu.VMEM((16,), jnp.int32),
        pltpu.VMEM((RNG + 8, D), jnp.float32),
        pltpu.VMEM((GW, D), jnp.float32),
        pltpu.VMEM((GW, D), jnp.float32),
        pltpu.SemaphoreType.DMA,
        pltpu.SemaphoreType.DMA,
    ],
)
def _segmax(a_hbm, sel_hbm, cnt_hbm, out_hbm,
            sell, srcl, cntv, acc, rows0, rows1, sem0, sem1):
    wid = lax.axis_index("s") * 2 + lax.axis_index("c")
    n0 = wid * RNG
    pltpu.async_copy(a_hbm.at[pl.ds(n0, RNG)], acc.at[pl.ds(0, RNG)], sem1)
    pltpu.sync_copy(sel_hbm.at[wid], sell)
    pltpu.sync_copy(cnt_hbm.at[wid], cntv)
    nwin2 = cntv[...][0] // (2 * GW)

    def src_body(g, c):
        srcl[pl.ds(g * 16, 16)] = sell[pl.ds(g * 16, 16)] >> 9
        return c

    lax.fori_loop(0, nwin2 * (2 * GW // 16), src_body, 0)
    pltpu.make_async_copy(a_hbm.at[pl.ds(0, RNG)], acc.at[pl.ds(0, RNG)],
                          sem1).wait()

    def compute(rows, base):
        def grp_body(g, c2):
            d16 = sell[pl.ds(base + g * 16, 16)] & 511
            for j in range(16):
                dloc = d16[j]
                for f in range(D // 16):
                    sl = pl.ds(f * 16, 16)
                    acc[dloc, sl] = jnp.maximum(acc[dloc, sl],
                                                rows[g * 16 + j, sl])
            return c2

        lax.fori_loop(0, GW // 16, grp_body, 0)

    pltpu.async_copy(a_hbm.at[srcl.at[pl.ds(0, GW)]], rows0, sem0)

    def win_body(k, carry):
        base = k * 2 * GW
        pltpu.async_copy(a_hbm.at[srcl.at[pl.ds(base + GW, GW)]], rows1, sem1)
        pltpu.make_async_copy(a_hbm.at[pl.ds(0, GW)], rows0, sem0).wait()
        compute(rows0, base)

        @pl.when(k + 1 < nwin2)
        def _():
            pltpu.async_copy(a_hbm.at[srcl.at[pl.ds(base + 2 * GW, GW)]],
                             rows0, sem0)

        pltpu.make_async_copy(a_hbm.at[pl.ds(0, GW)], rows1, sem1).wait()
        compute(rows1, base + GW)
        return carry

    lax.fori_loop(0, nwin2, win_body, 0)
    pltpu.sync_copy(acc.at[pl.ds(0, RNG)], out_hbm.at[pl.ds(n0, RNG)])


def _lin_body(xc_ref, w_ref, o_ref):
    o_ref[...] = jnp.dot(xc_ref[...], w_ref[...],
                         preferred_element_type=jnp.float32)


_lin = pl.pallas_call(
    _lin_body,
    grid=(NB,),
    in_specs=[
        pl.BlockSpec((1280, 2 * D), lambda i: (i, 0)),
        pl.BlockSpec((2 * D, 2 * D), lambda i: (0, 0)),
    ],
    out_specs=pl.BlockSpec((1280, 2 * D), lambda i: (i, 0)),
    out_shape=jax.ShapeDtypeStruct((N2, 2 * D), jnp.float32),
)


def _mid_body(s_ref, p_ref, w_ref, y_ref, sm_ref, sq_ref):
    t = s_ref[...] - p_ref[...]
    y = jnp.dot(t, w_ref[...], preferred_element_type=jnp.float32)
    rows = jax.lax.broadcasted_iota(jnp.int32, y.shape, 0) + pl.program_id(0) * 1280
    y = jnp.where(rows < N, y, 0.0)
    y_ref[...] = y
    sm_ref[...] = jnp.sum(y, axis=0, keepdims=True)[None]
    sq_ref[...] = jnp.sum(y * y, axis=0, keepdims=True)[None]


_mid = pl.pallas_call(
    _mid_body,
    grid=(NB,),
    in_specs=[
        pl.BlockSpec((1280, D), lambda i: (i, 0)),
        pl.BlockSpec((1280, D), lambda i: (i, 0)),
        pl.BlockSpec((D, D), lambda i: (0, 0)),
    ],
    out_specs=[
        pl.BlockSpec((1280, D), lambda i: (i, 0)),
        pl.BlockSpec((1, 1, D), lambda i: (i, 0, 0)),
        pl.BlockSpec((1, 1, D), lambda i: (i, 0, 0)),
    ],
    out_shape=[
        jax.ShapeDtypeStruct((N2, D), jnp.float32),
        jax.ShapeDtypeStruct((NB, 1, D), jnp.float32),
        jax.ShapeDtypeStruct((NB, 1, D), jnp.float32),
    ],
)


def _apply1_body(y_ref, sm_ref, sq_ref, g_ref, b_ref, pos2f_ref,
                 w21a_ref, w21bp_ref, a2_ref, p2_ref):
    m = jnp.sum(sm_ref[...], axis=0) / N
    v = jnp.sum(sq_ref[...], axis=0) / N - m * m
    s = g_ref[...] * jax.lax.rsqrt(v + 1e-5)
    h = jnp.maximum(y_ref[...] * s + (b_ref[...] - m * s), 0.0)
    p2 = jnp.dot(pos2f_ref[...], w21bp_ref[...],
                 preferred_element_type=jnp.float32)
    a2_ref[...] = jnp.dot(h, w21a_ref[...],
                          preferred_element_type=jnp.float32) + p2
    p2_ref[...] = p2


_apply1 = pl.pallas_call(
    _apply1_body,
    grid=(NB,),
    in_specs=[
        pl.BlockSpec((1280, D), lambda i: (i, 0)),
        pl.BlockSpec((NB, 1, D), lambda i: (0, 0, 0)),
        pl.BlockSpec((NB, 1, D), lambda i: (0, 0, 0)),
        pl.BlockSpec((1, D), lambda i: (0, 0)),
        pl.BlockSpec((1, D), lambda i: (0, 0)),
        pl.BlockSpec((1280, D), lambda i: (i, 0)),
        pl.BlockSpec((D, D), lambda i: (0, 0)),
        pl.BlockSpec((D, D), lambda i: (0, 0)),
    ],
    out_specs=[
        pl.BlockSpec((1280, D), lambda i: (i, 0)),
        pl.BlockSpec((1280, D), lambda i: (i, 0)),
    ],
    out_shape=[
        jax.ShapeDtypeStruct((N2, D), jnp.float32),
        jax.ShapeDtypeStruct((N2, D), jnp.float32),
    ],
)


def _apply2_body(y_ref, sm_ref, sq_ref, g_ref, b_ref, x_ref, o_ref):
    m = jnp.sum(sm_ref[...], axis=0) / N
    v = jnp.sum(sq_ref[...], axis=0) / N - m * m
    s = g_ref[...] * jax.lax.rsqrt(v + 1e-5)
    o_ref[...] = jnp.maximum(y_ref[...] * s + (b_ref[...] - m * s) + x_ref[...],
                             0.0)


_apply2 = pl.pallas_call(
    _apply2_body,
    grid=(NB,),
    in_specs=[
        pl.BlockSpec((1280, D), lambda i: (i, 0)),
        pl.BlockSpec((NB, 1, D), lambda i: (0, 0, 0)),
        pl.BlockSpec((NB, 1, D), lambda i: (0, 0, 0)),
        pl.BlockSpec((1, D), lambda i: (0, 0)),
        pl.BlockSpec((1, D), lambda i: (0, 0)),
        pl.BlockSpec((1280, D), lambda i: (i, 0)),
    ],
    out_specs=pl.BlockSpec((1280, D), lambda i: (i, 0)),
    out_shape=jax.ShapeDtypeStruct((N2, D), jnp.float32),
)


def kernel(x, pos, edge_index, W11, W12, g1, b1, W21, W22, g2, b2):
    pos2 = pos[:, :2]
    xc = jnp.zeros((N2, 2 * D), jnp.float32)
    xc = xc.at[:N, :D].set(x).at[:N, D:D + 2].set(pos2)
    pos2f = jnp.zeros((N2, D), jnp.float32).at[:N, :2].set(pos2)
    xpad = jnp.zeros((N2, D), jnp.float32).at[:N].set(x)
    Wc1 = jnp.zeros((2 * D, 2 * D), jnp.float32)
    Wc1 = Wc1.at[:D + 2, :D].set(W11).at[D:D + 2, D:].set(W11[D:])
    w21bp = jnp.zeros((D, D), jnp.float32).at[:2].set(W21[D:])
    w21a = W21[:D]
    g1r, b1r = g1.reshape(1, D), b1.reshape(1, D)
    g2r, b2r = g2.reshape(1, D), b2.reshape(1, D)

    srcp = jnp.concatenate([edge_index[0],
                            jnp.zeros((E2 - E,), jnp.int32)])
    dstp = jnp.concatenate([edge_index[1],
                            jnp.full((E2 - E,), 1 << 20, jnp.int32)])

    sel, cnts = _select(srcp, dstp)

    ap1 = _lin(xc, Wc1)
    a1 = ap1[:, :D]
    p1 = ap1[:, D:]
    s1 = _segmax(a1, sel, cnts)
    y1, sm1, sq1 = _mid(s1, p1, W12)
    a2, p2 = _apply1(y1, sm1, sq1, g1r, b1r, pos2f, w21a, w21bp)
    s2 = _segmax(a2, sel, cnts)
    y2, sm2, sq2 = _mid(s2, p2, W22)
    out = _apply2(y2, sm2, sq2, g2r, b2r, xpad)
    return out[:N]
